# baseline (speedup 1.0000x reference)
import functools
import math

import numpy as np
import jax
import jax.numpy as jnp
from jax import lax
from jax.experimental import pallas as pl
from jax.experimental.pallas import tpu as pltpu
from jax.experimental.pallas import tpu_sc as plsc

VOCAB = 64
D = 512
SEQ = 512
BATCH = 128
NW = 32
POS_PER_W = SEQ // NW
HALF = BATCH // 2
NVEC = D // 16
SCALE = math.sqrt(float(D))


def _pos_encoding_np():
    depth = D / 2
    positions = np.arange(SEQ)[:, np.newaxis]
    depths = np.arange(depth)[np.newaxis, :] / depth
    angle_rates = 1 / 10000 ** depths
    angle_rads = positions * angle_rates
    return np.concatenate([np.sin(angle_rads), np.cos(angle_rads)],
                          axis=-1).astype(np.float32)


_POS_NP = _pos_encoding_np()


def _sc_body(xt_hbm, pos_hbm, table_hbm, out_hbm,
             table_v, pos_v, x_v2, out_v0, out_v1, idx_v0, idx_v1,
             sem0, sem1):
    w = lax.axis_index("s") * 2 + lax.axis_index("c")
    s0 = w * POS_PER_W
    bufs = ((out_v0, idx_v0, sem0), (out_v1, idx_v1, sem1))

    pltpu.sync_copy(table_hbm, table_v)
    def scale_row(r, _):
        for j in range(NVEC):
            sl = pl.ds(16 * j, 16)
            table_v[r, sl] = table_v[r, sl] * SCALE
        return 0
    lax.fori_loop(0, VOCAB, scale_row, 0)

    pltpu.sync_copy(pos_hbm.at[pl.ds(s0, POS_PER_W)], pos_v)
    pltpu.sync_copy(xt_hbm.at[:, pl.ds(128 * (w // 8), 128)],
                    x_v2.at[pl.ds(0, BATCH)])
    woff = s0 % 128

    lane = lax.iota(jnp.int32, 16)

    def pos_body(p, _):
        s = s0 + p
        pj = [pos_v[p, pl.ds(16 * j, 16)] for j in range(NVEC)]
        for h in range(2):
            out_v, idx_v, sem = bufs[h]
            b0 = h * HALF

            @pl.when(p > 0)
            def _():
                pltpu.make_async_copy(out_v, out_hbm.at[idx_v], sem).wait()

            @plsc.parallel_loop(0, HALF, unroll=4)
            def _(t):
                xv = x_v2[b0 + t, pl.ds(woff + p, 16)]
                v = xv[0]
                for j in range(NVEC):
                    sl = pl.ds(16 * j, 16)
                    out_v[t, sl] = table_v[v, sl] + pj[j]

            for c in range(4):
                idx_v[pl.ds(16 * c, 16)] = (lane + (b0 + 16 * c)) * SEQ + s
            pltpu.async_copy(out_v, out_hbm.at[idx_v], sem)
        return 0
    lax.fori_loop(0, POS_PER_W, pos_body, 0)

    for h in range(2):
        out_v, idx_v, sem = bufs[h]
        pltpu.make_async_copy(out_v, out_hbm.at[idx_v], sem).wait()


@jax.jit
def _sc_call(xt, pos, table):
    kfn = pl.kernel(
        _sc_body,
        out_type=jax.ShapeDtypeStruct((BATCH * SEQ, D), jnp.float32),
        mesh=plsc.VectorSubcoreMesh(core_axis_name="c", subcore_axis_name="s"),
        scratch_types=[
            pltpu.VMEM((VOCAB, D), jnp.float32),
            pltpu.VMEM((POS_PER_W, D), jnp.float32),
            pltpu.VMEM((BATCH + 1, 128), jnp.int32),
            pltpu.VMEM((HALF, D), jnp.float32),
            pltpu.VMEM((HALF, D), jnp.float32),
            pltpu.VMEM((HALF,), jnp.int32),
            pltpu.VMEM((HALF,), jnp.int32),
            pltpu.SemaphoreType.DMA,
            pltpu.SemaphoreType.DMA,
        ],
    )
    return kfn(xt, pos, table)


def kernel(x, table):
    out = _sc_call(x.astype(jnp.int32), jnp.asarray(_POS_NP), table)
    return out.reshape(BATCH, SEQ, D)


# speedup vs baseline: 1.2861x; 1.2861x over previous
import functools
import math

import numpy as np
import jax
import jax.numpy as jnp
from jax import lax
from jax.experimental import pallas as pl
from jax.experimental.pallas import tpu as pltpu
from jax.experimental.pallas import tpu_sc as plsc

VOCAB = 64
D = 512
SEQ = 512
BATCH = 128
NW = 32
POS_PER_W = SEQ // NW
HALF = BATCH // 2
NVEC = D // 16
SCALE = math.sqrt(float(D))


def _pos_encoding_np():
    depth = D / 2
    positions = np.arange(SEQ)[:, np.newaxis]
    depths = np.arange(depth)[np.newaxis, :] / depth
    angle_rates = 1 / 10000 ** depths
    angle_rads = positions * angle_rates
    return np.concatenate([np.sin(angle_rads), np.cos(angle_rads)],
                          axis=-1).astype(np.float32)


_POS_NP = _pos_encoding_np()


def _sc_body(xt_hbm, pos_hbm, table_hbm, out_hbm,
             table_v, pos_v, x_v2, out_v0, out_v1, idx_v0, idx_v1,
             sem0, sem1):
    w = lax.axis_index("s") * 2 + lax.axis_index("c")
    s0 = w * POS_PER_W
    bufs = ((out_v0, idx_v0, sem0), (out_v1, idx_v1, sem1))

    pltpu.sync_copy(table_hbm, table_v)
    def scale_row(r, _):
        for j in range(NVEC):
            sl = pl.ds(16 * j, 16)
            table_v[r, sl] = table_v[r, sl] * SCALE
        return 0
    lax.fori_loop(0, VOCAB, scale_row, 0)

    pltpu.sync_copy(pos_hbm.at[pl.ds(s0, POS_PER_W)], pos_v)
    pltpu.sync_copy(xt_hbm.at[:, pl.ds(128 * (w // 8), 128)],
                    x_v2.at[pl.ds(0, BATCH)])
    woff = s0 % 128

    lane = lax.iota(jnp.int32, 16)

    def pos_body(p, _):
        s = s0 + p
        pj = [pos_v[p, pl.ds(16 * j, 16)] for j in range(NVEC)]
        for h in range(2):
            out_v, idx_v, sem = bufs[h]
            b0 = h * HALF

            @pl.when(p > 0)
            def _():
                pltpu.make_async_copy(out_v, out_hbm.at[idx_v], sem).wait()

            @plsc.parallel_loop(0, HALF, unroll=1)
            def _(t):
                xv = x_v2[b0 + t, pl.ds(woff + p, 16)]
                v = xv[0]
                for j in range(NVEC):
                    sl = pl.ds(16 * j, 16)
                    out_v[t, sl] = table_v[v, sl] + pj[j]

            for c in range(4):
                idx_v[pl.ds(16 * c, 16)] = (lane + (b0 + 16 * c)) * SEQ + s
            pltpu.async_copy(out_v, out_hbm.at[idx_v], sem)
        return 0
    lax.fori_loop(0, POS_PER_W, pos_body, 0)

    for h in range(2):
        out_v, idx_v, sem = bufs[h]
        pltpu.make_async_copy(out_v, out_hbm.at[idx_v], sem).wait()


@jax.jit
def _sc_call(xt, pos, table):
    kfn = pl.kernel(
        _sc_body,
        out_type=jax.ShapeDtypeStruct((BATCH * SEQ, D), jnp.float32),
        mesh=plsc.VectorSubcoreMesh(core_axis_name="c", subcore_axis_name="s"),
        scratch_types=[
            pltpu.VMEM((VOCAB, D), jnp.float32),
            pltpu.VMEM((POS_PER_W, D), jnp.float32),
            pltpu.VMEM((BATCH + 1, 128), jnp.int32),
            pltpu.VMEM((HALF, D), jnp.float32),
            pltpu.VMEM((HALF, D), jnp.float32),
            pltpu.VMEM((HALF,), jnp.int32),
            pltpu.VMEM((HALF,), jnp.int32),
            pltpu.SemaphoreType.DMA,
            pltpu.SemaphoreType.DMA,
        ],
    )
    return kfn(xt, pos, table)


def kernel(x, table):
    out = _sc_call(x.astype(jnp.int32), jnp.asarray(_POS_NP), table)
    return out.reshape(BATCH, SEQ, D)
